# baseline (speedup 1.0000x reference)
import jax
import jax.numpy as jnp
from jax.experimental import pallas as pl
from jax.experimental.pallas import tpu as pltpu

_B = 2
_D = 64
_L2_LO, _L2_N = 256, 128
_L3_LO, _L3_N = 384, 10
_EPB = _L2_N * _L3_N
_E = _B * _EPB


def _gnn_kernel(nodes_ref, e_ref, Wn_ref, bn_ref, We_ref, be_ref,
                Wm_ref, bm_ref, Wu_ref, bu_ref, W1_ref, b1_ref,
                W2_ref, b2_ref, W3_ref, b3_ref, out_ref):
    d = _D
    f32 = jnp.float32
    Wn = Wn_ref[...]
    Wm1 = Wm_ref[0:d, :]
    Wm2 = Wm_ref[d:2 * d, :]
    Wm3 = Wm_ref[2 * d:3 * d, :]
    Wu1 = Wu_ref[0:d, :]
    Wu2 = Wu_ref[d:2 * d, :]
    bn = bn_ref[...].reshape(1, d)
    be = be_ref[...].reshape(1, d)
    bm = bm_ref[...].reshape(1, d)
    bu = bu_ref[...].reshape(1, d)
    b1 = b1_ref[...].reshape(1, d)
    b2 = b2_ref[...].reshape(1, d)
    b3 = b3_ref[...].reshape(1, _L3_N)

    def mm(a, b):
        return jnp.dot(a, b, preferred_element_type=f32)

    A1 = mm(Wn, Wm1)
    A2 = mm(Wn, Wm2)
    A3 = mm(We_ref[...], Wm3)
    const = mm(bn, Wm1) + mm(bn, Wm2) + mm(be, Wm3) + bm

    n2 = jnp.concatenate([nodes_ref[b, pl.ds(_L2_LO, _L2_N), :]
                          for b in range(_B)], axis=0)
    n3 = jnp.concatenate([nodes_ref[b, pl.ds(_L3_LO, _L3_N), :]
                          for b in range(_B)], axis=0)
    xs2 = mm(n2, A1)
    xd3 = mm(n3, A2)

    r = jax.lax.broadcasted_iota(jnp.int32, (_E, 1), 0)
    bix = r // _EPB
    q = r - bix * _EPB
    src_id = bix * _L2_N + q // _L3_N
    dst_id = bix * _L3_N + (q - (q // _L3_N) * _L3_N)
    Q = (jax.lax.broadcasted_iota(jnp.int32, (1, _B * _L2_N), 1)
         == src_id).astype(f32)
    P = (jax.lax.broadcasted_iota(jnp.int32, (1, _B * _L3_N), 1)
         == dst_id).astype(f32)

    ea = mm(e_ref[...], A3)
    msg = jax.nn.relu(ea + mm(Q, xs2) + mm(P, xd3) + const)
    agg = jax.lax.dot_general(P, msg, (((0,), (0,)), ((), ())),
                              preferred_element_type=f32)

    x3 = mm(n3, Wn) + bn
    node = jax.nn.relu(mm(x3, Wu1) + mm(agg, Wu2) + bu)
    G = ((jax.lax.broadcasted_iota(jnp.int32, (_B, _B * _L3_N), 1) // _L3_N
          == jax.lax.broadcasted_iota(jnp.int32, (_B, _B * _L3_N), 0))
         .astype(f32) / _L3_N)
    gf = mm(G, node)
    h = jax.nn.relu(mm(gf, W1_ref[...]) + b1)
    h = jax.nn.relu(mm(h, W2_ref[...]) + b2)
    out_ref[...] = mm(h, W3_ref[...]) + b3


def kernel(inputs_nodes, inputs_edges, Wn, bn, We, be, Wm, bm, Wu, bu,
           W1, b1, W2, b2, W3, b3):
    e_blk = jax.lax.slice(inputs_edges,
                          (0, _L2_LO, _L3_LO, 0),
                          (_B, _L2_LO + _L2_N, _L3_LO + _L3_N, _D))
    e_flat = e_blk.reshape(_E, _D)
    vmem = pl.BlockSpec(memory_space=pltpu.MemorySpace.VMEM)
    return pl.pallas_call(
        _gnn_kernel,
        out_shape=jax.ShapeDtypeStruct((_B, _L3_N), jnp.float32),
        in_specs=[vmem] * 16,
        out_specs=vmem,
    )(inputs_nodes, e_flat, Wn, bn, We, be, Wm, bm, Wu, bu,
      W1, b1, W2, b2, W3, b3)
